# Initial kernel scaffold; baseline (speedup 1.0000x reference)
import jax
import jax.numpy as jnp
from jax.experimental import pallas as pl


def kernel(spans, span_indices, label_lst, target_indices, opinion_indices, distance_embeddings):
    raise NotImplementedError("write your pallas kernel here")


# speedup vs baseline: 4.0533x; 4.0533x over previous
import functools

import jax
import jax.numpy as jnp
from jax import lax
from jax.experimental import pallas as pl
from jax.experimental.pallas import tpu as pltpu
from jax.experimental.pallas import tpu_sc as plsc

_BINS = (0, 1, 2, 3, 4, 5, 7, 8, 15, 16, 31, 32, 63, 64)


def kernel(spans, span_indices, label_lst, target_indices, opinion_indices, distance_embeddings):
    B, N, H = spans.shape
    T = target_indices.shape[1]
    O = opinion_indices.shape[1]
    P = T * O
    D = distance_embeddings.shape[1]
    W = 2 * H + D

    spans2d = spans.reshape(B * N, H)
    t_flat = target_indices.reshape(-1)
    o_flat = opinion_indices.reshape(-1)
    span_start = span_indices[:, 0]
    span_end = span_indices[:, 1]

    mesh = plsc.VectorSubcoreMesh(core_axis_name="c", subcore_axis_name="s")
    TH = T // 2

    @functools.partial(
        pl.kernel,
        mesh=mesh,
        compiler_params=pltpu.CompilerParams(needs_layout_passes=False),
        out_type=(
            jax.ShapeDtypeStruct((B * P, W), jnp.float32),
            jax.ShapeDtypeStruct((B * P * 2,), jnp.int32),
        ),
        scratch_types=[
            pltpu.VMEM((TH,), jnp.int32),
            pltpu.VMEM((O,), jnp.int32),
            pltpu.VMEM((O,), jnp.int32),
            pltpu.VMEM((O,), jnp.int32),
            pltpu.VMEM((N,), jnp.int32),
            pltpu.VMEM((N,), jnp.int32),
            pltpu.VMEM((O, H), jnp.float32),
            pltpu.VMEM((O, H), jnp.float32),
            pltpu.VMEM((O, D), jnp.float32),
            pltpu.VMEM((O,), jnp.int32),
            pltpu.VMEM((O * 2,), jnp.int32),
            pltpu.SemaphoreType.DMA,
        ],
    )
    def sc_kern(spans_hbm, sstart_hbm, send_hbm, tflat_hbm, oflat_hbm, demb_hbm,
                out_hbm, rel_hbm,
                tind_v, oind_v, oidx_v, trep_v, sstart_v, send_v,
                so_v, tt_v, dist_v, em_v, rel_v, sem):
        b = lax.axis_index("s")
        h = lax.axis_index("c")
        t0 = h * TH
        base = b * N

        pltpu.sync_copy(tflat_hbm.at[pl.ds(b * T + t0, TH)], tind_v)
        pltpu.sync_copy(oflat_hbm.at[pl.ds(b * O, O)], oind_v)
        pltpu.sync_copy(sstart_hbm, sstart_v)
        pltpu.sync_copy(send_hbm, send_v)

        for half in range(O // 16):
            oidx_v[pl.ds(half * 16, 16)] = oind_v[pl.ds(half * 16, 16)] + base
        pltpu.async_copy(spans_hbm.at[oidx_v], so_v, sem).wait()

        iota = lax.iota(jnp.int32, 16)

        o_raw = [oind_v[pl.ds(half * 16, 16)] for half in range(O // 16)]
        c_o = [plsc.load_gather(sstart_v, [o_raw[half]]) for half in range(O // 16)]
        d_o = [plsc.load_gather(send_v, [o_raw[half]]) for half in range(O // 16)]

        tvec = tind_v[pl.ds(0, 16)]
        for t in range(TH):
            tt_s = jnp.sum(tvec * (iota == t).astype(jnp.int32))
            tt_raw = jnp.zeros((16,), jnp.int32) + tt_s
            a_t = plsc.load_gather(sstart_v, [tt_raw])
            b_t = plsc.load_gather(send_v, [tt_raw])
            tt_g = tt_raw + base
            for half in range(O // 16):
                trep_v[pl.ds(half * 16, 16)] = tt_g
                width = jnp.minimum(jnp.abs(b_t - c_o[half]), jnp.abs(a_t - d_o[half]))
                em = jnp.full((16,), -1, jnp.int32)
                for edge in _BINS:
                    em = em + (width >= edge).astype(jnp.int32)
                em_v[pl.ds(half * 16, 16)] = em
                slots = jnp.full((16,), half * 32, jnp.int32) + iota * 2
                plsc.store_scatter(rel_v, [slots], tt_raw)
                plsc.store_scatter(rel_v, [slots + 1], o_raw[half])
            pltpu.async_copy(spans_hbm.at[trep_v], tt_v, sem).wait()
            pltpu.async_copy(demb_hbm.at[em_v], dist_v, sem).wait()
            row0 = (b * T + t0 + t) * O
            pltpu.sync_copy(tt_v, out_hbm.at[pl.ds(row0, O), pl.ds(0, H)])
            pltpu.sync_copy(so_v, out_hbm.at[pl.ds(row0, O), pl.ds(H, H)])
            pltpu.sync_copy(dist_v, out_hbm.at[pl.ds(row0, O), pl.ds(2 * H, D)])
            pltpu.sync_copy(rel_v, rel_hbm.at[pl.ds(row0 * 2, O * 2)])

    out2d, relflat = sc_kern(spans2d, span_start, span_end, t_flat, o_flat,
                             distance_embeddings)
    return out2d.reshape(B, P, W), relflat.reshape(B, P, 2)


# speedup vs baseline: 4.0618x; 1.0021x over previous
import functools

import jax
import jax.numpy as jnp
from jax import lax
from jax.experimental import pallas as pl
from jax.experimental.pallas import tpu as pltpu
from jax.experimental.pallas import tpu_sc as plsc

_BINS = (0, 1, 2, 3, 4, 5, 7, 8, 15, 16, 31, 32, 63, 64)


def kernel(spans, span_indices, label_lst, target_indices, opinion_indices, distance_embeddings):
    B, N, H = spans.shape
    T = target_indices.shape[1]
    O = opinion_indices.shape[1]
    P = T * O
    D = distance_embeddings.shape[1]
    W = 2 * H + D

    spans2d = spans.reshape(B * N, H)
    t_flat = target_indices.reshape(-1)
    o_flat = opinion_indices.reshape(-1)
    span_start = span_indices[:, 0]
    span_end = span_indices[:, 1]

    mesh = plsc.VectorSubcoreMesh(core_axis_name="c", subcore_axis_name="s")
    TH = T // 2

    @functools.partial(
        pl.kernel,
        mesh=mesh,
        compiler_params=pltpu.CompilerParams(needs_layout_passes=False),
        out_type=(
            jax.ShapeDtypeStruct((B * P, W), jnp.float32),
            jax.ShapeDtypeStruct((B * P * 2,), jnp.int32),
        ),
        scratch_types=[
            pltpu.VMEM((TH,), jnp.int32),
            pltpu.VMEM((O,), jnp.int32),
            pltpu.VMEM((O,), jnp.int32),
            pltpu.VMEM((N,), jnp.int32),
            pltpu.VMEM((N,), jnp.int32),
            pltpu.VMEM((O, H), jnp.float32),
            pltpu.VMEM((2, O), jnp.int32),
            pltpu.VMEM((2, O, H), jnp.float32),
            pltpu.VMEM((2, O, D), jnp.float32),
            pltpu.VMEM((2, O), jnp.int32),
            pltpu.VMEM((2, O * 2), jnp.int32),
            pltpu.SemaphoreType.DMA,
            pltpu.SemaphoreType.DMA,
            pltpu.SemaphoreType.DMA,
            pltpu.SemaphoreType.DMA,
            pltpu.SemaphoreType.DMA,
        ],
    )
    def sc_kern(spans_hbm, sstart_hbm, send_hbm, tflat_hbm, oflat_hbm, demb_hbm,
                out_hbm, rel_hbm,
                tind_v, oind_v, oidx_v, sstart_v, send_v,
                so_v, trep_v, tt_v, dist_v, em_v, rel_v,
                sem_in, gsem0, gsem1, osem0, osem1):
        b = lax.axis_index("s")
        h = lax.axis_index("c")
        t0 = h * TH
        base = b * N

        h_in = [
            pltpu.async_copy(tflat_hbm.at[pl.ds(b * T + t0, TH)], tind_v, sem_in),
            pltpu.async_copy(oflat_hbm.at[pl.ds(b * O, O)], oind_v, sem_in),
            pltpu.async_copy(sstart_hbm, sstart_v, sem_in),
            pltpu.async_copy(send_hbm, send_v, sem_in),
        ]
        for hd in h_in:
            hd.wait()

        for half in range(O // 16):
            oidx_v[pl.ds(half * 16, 16)] = oind_v[pl.ds(half * 16, 16)] + base
        pltpu.async_copy(spans_hbm.at[oidx_v], so_v, sem_in).wait()

        iota = lax.iota(jnp.int32, 16)
        o_raw = [oind_v[pl.ds(half * 16, 16)] for half in range(O // 16)]
        c_o = [plsc.load_gather(sstart_v, [o_raw[half]]) for half in range(O // 16)]
        d_o = [plsc.load_gather(send_v, [o_raw[half]]) for half in range(O // 16)]
        tvec = tind_v[pl.ds(0, 16)]

        gsems = (gsem0, gsem1)
        osems = (osem0, osem1)
        g_handles = [[], []]
        out_handles = [[], []]

        def stage(t):
            k = t % 2
            for hd in out_handles[k]:
                hd.wait()
            out_handles[k] = []
            tt_s = jnp.sum(tvec * (iota == t).astype(jnp.int32))
            tt_raw = jnp.zeros((16,), jnp.int32) + tt_s
            a_t = plsc.load_gather(sstart_v, [tt_raw])
            b_t = plsc.load_gather(send_v, [tt_raw])
            tt_g = tt_raw + base
            for half in range(O // 16):
                trep_v[t % 2, pl.ds(half * 16, 16)] = tt_g
                width = jnp.minimum(jnp.abs(b_t - c_o[half]), jnp.abs(a_t - d_o[half]))
                em = jnp.full((16,), -1, jnp.int32)
                for edge in _BINS:
                    em = em + (width >= edge).astype(jnp.int32)
                em_v[k, pl.ds(half * 16, 16)] = em
                slots = jnp.full((16,), half * 32, jnp.int32) + iota * 2
                plsc.store_scatter(rel_v.at[k], [slots], tt_raw)
                plsc.store_scatter(rel_v.at[k], [slots + 1], o_raw[half])
            g_handles[k] = [
                pltpu.async_copy(spans_hbm.at[trep_v.at[k]], tt_v.at[k], gsems[k]),
                pltpu.async_copy(demb_hbm.at[em_v.at[k]], dist_v.at[k], gsems[k]),
            ]

        stage(0)
        for t in range(TH):
            k = t % 2
            if t + 1 < TH:
                stage(t + 1)
            for hd in g_handles[k]:
                hd.wait()
            g_handles[k] = []
            row0 = (b * T + t0 + t) * O
            out_handles[k] = [
                pltpu.async_copy(tt_v.at[k], out_hbm.at[pl.ds(row0, O), pl.ds(0, H)], osems[k]),
                pltpu.async_copy(so_v, out_hbm.at[pl.ds(row0, O), pl.ds(H, H)], osems[k]),
                pltpu.async_copy(dist_v.at[k], out_hbm.at[pl.ds(row0, O), pl.ds(2 * H, D)], osems[k]),
                pltpu.async_copy(rel_v.at[k], rel_hbm.at[pl.ds(row0 * 2, O * 2)], osems[k]),
            ]
        for k in range(2):
            for hd in out_handles[k]:
                hd.wait()

    out2d, relflat = sc_kern(spans2d, span_start, span_end, t_flat, o_flat,
                             distance_embeddings)
    return out2d.reshape(B, P, W), relflat.reshape(B, P, 2)


# speedup vs baseline: 4.9162x; 1.2104x over previous
import functools

import jax
import jax.numpy as jnp
from jax import lax
from jax.experimental import pallas as pl
from jax.experimental.pallas import tpu as pltpu
from jax.experimental.pallas import tpu_sc as plsc

_BINS = (0, 1, 2, 3, 4, 5, 7, 8, 15, 16, 31, 32, 63, 64)


def kernel(spans, span_indices, label_lst, target_indices, opinion_indices, distance_embeddings):
    B, N, H = spans.shape
    T = target_indices.shape[1]
    O = opinion_indices.shape[1]
    P = T * O
    D = distance_embeddings.shape[1]
    W = 2 * H + D
    NB = 16

    spans2d = spans.reshape(B * N, H)
    t_flat = target_indices.reshape(-1)
    o_flat = opinion_indices.reshape(-1)
    span_start = span_indices[:, 0]
    span_end = span_indices[:, 1]
    demb16 = jnp.zeros((NB, D), jnp.float32).at[:14].set(distance_embeddings)

    mesh = plsc.VectorSubcoreMesh(core_axis_name="c", subcore_axis_name="s")
    TH = T // 2

    @functools.partial(
        pl.kernel,
        mesh=mesh,
        compiler_params=pltpu.CompilerParams(needs_layout_passes=False),
        out_type=(
            jax.ShapeDtypeStruct((B * (T + O), H), jnp.float32),
            jax.ShapeDtypeStruct((B * P * NB,), jnp.float32),
            jax.ShapeDtypeStruct((B * P * 2,), jnp.int32),
        ),
        scratch_types=[
            pltpu.VMEM((TH,), jnp.int32),
            pltpu.VMEM((O,), jnp.int32),
            pltpu.VMEM((TH,), jnp.int32),
            pltpu.VMEM((TH,), jnp.int32),
            pltpu.VMEM((N,), jnp.int32),
            pltpu.VMEM((N,), jnp.int32),
            pltpu.VMEM((TH, H), jnp.float32),
            pltpu.VMEM((TH, H), jnp.float32),
            pltpu.VMEM((2, O * NB), jnp.float32),
            pltpu.VMEM((2, O * 2), jnp.int32),
            pltpu.SemaphoreType.DMA,
            pltpu.SemaphoreType.DMA,
            pltpu.SemaphoreType.DMA,
        ],
    )
    def sc_kern(spans_hbm, sstart_hbm, send_hbm, tflat_hbm, oflat_hbm,
                compact_hbm, emoh_hbm, rel_hbm,
                tind_v, oind_v, tidx_v, oidx_v, sstart_v, send_v,
                trows_v, orows_v, emoh_v, rel_v,
                sem_in, osem0, osem1):
        b = lax.axis_index("s")
        h = lax.axis_index("c")
        t0 = h * TH
        base = b * N

        h_in = [
            pltpu.async_copy(tflat_hbm.at[pl.ds(b * T + t0, TH)], tind_v, sem_in),
            pltpu.async_copy(oflat_hbm.at[pl.ds(b * O, O)], oind_v, sem_in),
            pltpu.async_copy(sstart_hbm, sstart_v, sem_in),
            pltpu.async_copy(send_hbm, send_v, sem_in),
        ]
        for hd in h_in:
            hd.wait()

        iota = lax.iota(jnp.int32, 16)
        tidx_v[pl.ds(0, 16)] = tind_v[pl.ds(0, 16)] + base
        oidx_v[pl.ds(0, 16)] = oind_v[pl.ds(t0, 16)] + base
        g1 = pltpu.async_copy(spans_hbm.at[tidx_v], trows_v, sem_in)
        g2 = pltpu.async_copy(spans_hbm.at[oidx_v], orows_v, sem_in)

        o_raw = [oind_v[pl.ds(half * 16, 16)] for half in range(O // 16)]
        c_o = [plsc.load_gather(sstart_v, [o_raw[half]]) for half in range(O // 16)]
        d_o = [plsc.load_gather(send_v, [o_raw[half]]) for half in range(O // 16)]
        tvec = tind_v[pl.ds(0, 16)]

        osems = (osem0, osem1)
        out_handles = [[], []]
        for t in range(TH):
            k = t % 2
            for hd in out_handles[k]:
                hd.wait()
            out_handles[k] = []
            tt_s = jnp.sum(tvec * (iota == t).astype(jnp.int32))
            tt_raw = jnp.zeros((16,), jnp.int32) + tt_s
            a_t = plsc.load_gather(sstart_v, [tt_raw])
            b_t = plsc.load_gather(send_v, [tt_raw])
            for half in range(O // 16):
                width = jnp.minimum(jnp.abs(b_t - c_o[half]), jnp.abs(a_t - d_o[half]))
                em = jnp.full((16,), -1, jnp.int32)
                for edge in _BINS:
                    em = em + (width >= edge).astype(jnp.int32)
                for oo in range(16):
                    em_o = jnp.sum(em * (iota == oo).astype(jnp.int32))
                    row = (iota == em_o).astype(jnp.float32)
                    emoh_v[k, pl.ds((half * 16 + oo) * NB, NB)] = row
                slots = jnp.full((16,), half * 32, jnp.int32) + iota * 2
                plsc.store_scatter(rel_v.at[k], [slots], tt_raw)
                plsc.store_scatter(rel_v.at[k], [slots + 1], o_raw[half])
            row0 = (b * T + t0 + t) * O
            out_handles[k] = [
                pltpu.async_copy(emoh_v.at[k], emoh_hbm.at[pl.ds(row0 * NB, O * NB)], osems[k]),
                pltpu.async_copy(rel_v.at[k], rel_hbm.at[pl.ds(row0 * 2, O * 2)], osems[k]),
            ]
        for k in range(2):
            for hd in out_handles[k]:
                hd.wait()
        g1.wait()
        g2.wait()
        crow = b * (T + O)
        pltpu.sync_copy(trows_v, compact_hbm.at[pl.ds(crow + t0, TH)])
        pltpu.sync_copy(orows_v, compact_hbm.at[pl.ds(crow + T + t0, TH)])

    compact2d, emoh_flat, relflat = sc_kern(spans2d, span_start, span_end,
                                            t_flat, o_flat)
    compact = compact2d.reshape(B, T + O, H)
    emoh = emoh_flat.reshape(B * T, O, NB)

    def tc_body(compact_ref, emoh_ref, demb_ref, out_ref):
        s = pl.program_id(0)
        t = lax.rem(s, T)
        tgt = compact_ref[0, pl.ds(t, 1), :]
        out_ref[0, :, 0:H] = jnp.broadcast_to(tgt, (O, H))
        out_ref[0, :, H:2 * H] = compact_ref[0, T:T + O, :]
        oh = emoh_ref[0]
        out_ref[0, :, 2 * H:W] = jnp.dot(
            oh, demb_ref[...], preferred_element_type=jnp.float32)

    out3d = pl.pallas_call(
        tc_body,
        grid=(B * T,),
        in_specs=[
            pl.BlockSpec((1, T + O, H), lambda s: (s // T, 0, 0)),
            pl.BlockSpec((1, O, NB), lambda s: (s, 0, 0)),
            pl.BlockSpec((NB, D), lambda s: (0, 0)),
        ],
        out_specs=pl.BlockSpec((1, O, W), lambda s: (s, 0, 0)),
        out_shape=jax.ShapeDtypeStruct((B * T, O, W), jnp.float32),
        compiler_params=pltpu.CompilerParams(
            dimension_semantics=("arbitrary",),
        ),
    )(compact, emoh, demb16)

    return out3d.reshape(B, P, W), relflat.reshape(B, P, 2)


# speedup vs baseline: 19.1657x; 3.8985x over previous
import functools

import jax
import jax.numpy as jnp
from jax import lax
from jax.experimental import pallas as pl
from jax.experimental.pallas import tpu as pltpu
from jax.experimental.pallas import tpu_sc as plsc

_BINS = (0, 1, 2, 3, 4, 5, 7, 8, 15, 16, 31, 32, 63, 64)


def kernel(spans, span_indices, label_lst, target_indices, opinion_indices, distance_embeddings):
    B, N, H = spans.shape
    T = target_indices.shape[1]
    O = opinion_indices.shape[1]
    P = T * O
    D = distance_embeddings.shape[1]
    W = 2 * H + D
    NB = 16

    spans2d = spans.reshape(B * N, H)
    t_flat = target_indices.reshape(-1)
    o_flat = opinion_indices.reshape(-1)
    span_start = span_indices[:, 0]
    span_end = span_indices[:, 1]
    demb16 = jnp.zeros((NB, D), jnp.float32).at[:14].set(distance_embeddings)

    mesh = plsc.VectorSubcoreMesh(core_axis_name="c", subcore_axis_name="s")
    TH = T // 2

    @functools.partial(
        pl.kernel,
        mesh=mesh,
        compiler_params=pltpu.CompilerParams(needs_layout_passes=False),
        out_type=(
            jax.ShapeDtypeStruct((B * (T + O), H), jnp.float32),
            jax.ShapeDtypeStruct((B, P, NB), jnp.float32),
            jax.ShapeDtypeStruct((B * P * 2,), jnp.int32),
        ),
        scratch_types=[
            pltpu.VMEM((TH,), jnp.int32),
            pltpu.VMEM((O,), jnp.int32),
            pltpu.VMEM((TH,), jnp.int32),
            pltpu.VMEM((TH,), jnp.int32),
            pltpu.VMEM((N,), jnp.int32),
            pltpu.VMEM((N,), jnp.int32),
            pltpu.VMEM((TH, H), jnp.float32),
            pltpu.VMEM((TH, H), jnp.float32),
            pltpu.VMEM((2, O, NB), jnp.float32),
            pltpu.VMEM((2, O * 2), jnp.int32),
            pltpu.SemaphoreType.DMA,
            pltpu.SemaphoreType.DMA,
            pltpu.SemaphoreType.DMA,
        ],
    )
    def sc_kern(spans_hbm, sstart_hbm, send_hbm, tflat_hbm, oflat_hbm,
                compact_hbm, emoh_hbm, rel_hbm,
                tind_v, oind_v, tidx_v, oidx_v, sstart_v, send_v,
                trows_v, orows_v, emoh_v, rel_v,
                sem_in, osem0, osem1):
        b = lax.axis_index("s")
        h = lax.axis_index("c")
        t0 = h * TH
        base = b * N

        h_in = [
            pltpu.async_copy(tflat_hbm.at[pl.ds(b * T + t0, TH)], tind_v, sem_in),
            pltpu.async_copy(oflat_hbm.at[pl.ds(b * O, O)], oind_v, sem_in),
            pltpu.async_copy(sstart_hbm, sstart_v, sem_in),
            pltpu.async_copy(send_hbm, send_v, sem_in),
        ]
        for hd in h_in:
            hd.wait()

        iota = lax.iota(jnp.int32, 16)
        tidx_v[pl.ds(0, 16)] = tind_v[pl.ds(0, 16)] + base
        oidx_v[pl.ds(0, 16)] = oind_v[pl.ds(t0, 16)] + base
        g1 = pltpu.async_copy(spans_hbm.at[tidx_v], trows_v, sem_in)
        g2 = pltpu.async_copy(spans_hbm.at[oidx_v], orows_v, sem_in)

        o_raw = [oind_v[pl.ds(half * 16, 16)] for half in range(O // 16)]
        c_o = [plsc.load_gather(sstart_v, [o_raw[half]]) for half in range(O // 16)]
        d_o = [plsc.load_gather(send_v, [o_raw[half]]) for half in range(O // 16)]
        tvec = tind_v[pl.ds(0, 16)]

        osems = (osem0, osem1)
        out_handles = [[], []]
        for t in range(TH):
            k = t % 2
            for hd in out_handles[k]:
                hd.wait()
            out_handles[k] = []
            tt_s = jnp.sum(tvec * (iota == t).astype(jnp.int32))
            tt_raw = jnp.zeros((16,), jnp.int32) + tt_s
            a_t = plsc.load_gather(sstart_v, [tt_raw])
            b_t = plsc.load_gather(send_v, [tt_raw])
            for half in range(O // 16):
                width = jnp.minimum(jnp.abs(b_t - c_o[half]), jnp.abs(a_t - d_o[half]))
                em = jnp.full((16,), -1, jnp.int32)
                for edge in _BINS:
                    em = em + (width >= edge).astype(jnp.int32)
                for oo in range(16):
                    em_o = jnp.sum(em * (iota == oo).astype(jnp.int32))
                    emoh_v[k, half * 16 + oo, :] = (iota == em_o).astype(jnp.float32)
                slots = jnp.full((16,), half * 32, jnp.int32) + iota * 2
                plsc.store_scatter(rel_v.at[k], [slots], tt_raw)
                plsc.store_scatter(rel_v.at[k], [slots + 1], o_raw[half])
            p0 = (t0 + t) * O
            row0 = b * P + p0
            out_handles[k] = [
                pltpu.async_copy(emoh_v.at[k], emoh_hbm.at[b, pl.ds(p0, O), :], osems[k]),
                pltpu.async_copy(rel_v.at[k], rel_hbm.at[pl.ds(row0 * 2, O * 2)], osems[k]),
            ]
        for k in range(2):
            for hd in out_handles[k]:
                hd.wait()
        g1.wait()
        g2.wait()
        crow = b * (T + O)
        pltpu.sync_copy(trows_v, compact_hbm.at[pl.ds(crow + t0, TH)])
        pltpu.sync_copy(orows_v, compact_hbm.at[pl.ds(crow + T + t0, TH)])

    compact2d, emoh, relflat = sc_kern(spans2d, span_start, span_end,
                                       t_flat, o_flat)

    def tc_body(compact_ref, emoh_ref, demb_ref, out_ref):
        orows = compact_ref[T:T + O, :]
        for t in range(T):
            r0 = t * O
            tgt = compact_ref[pl.ds(t, 1), :]
            out_ref[0, r0:r0 + O, 0:H] = jnp.broadcast_to(tgt, (O, H))
            out_ref[0, r0:r0 + O, H:2 * H] = orows
        out_ref[0, :, 2 * H:W] = jnp.dot(
            emoh_ref[0], demb_ref[...], preferred_element_type=jnp.float32)

    out3d = pl.pallas_call(
        tc_body,
        grid=(B,),
        in_specs=[
            pl.BlockSpec((T + O, H), lambda s: (s, 0)),
            pl.BlockSpec((1, P, NB), lambda s: (s, 0, 0)),
            pl.BlockSpec((NB, D), lambda s: (0, 0)),
        ],
        out_specs=pl.BlockSpec((1, P, W), lambda s: (s, 0, 0)),
        out_shape=jax.ShapeDtypeStruct((B, P, W), jnp.float32),
        compiler_params=pltpu.CompilerParams(
            dimension_semantics=("arbitrary",),
        ),
    )(compact2d, emoh, demb16)

    return out3d, relflat.reshape(B, P, 2)


# speedup vs baseline: 22.7393x; 1.1865x over previous
import functools

import jax
import jax.numpy as jnp
from jax import lax
from jax.experimental import pallas as pl
from jax.experimental.pallas import tpu as pltpu
from jax.experimental.pallas import tpu_sc as plsc

_BINS = (0, 1, 2, 3, 4, 5, 7, 8, 15, 16, 31, 32, 63, 64)


def kernel(spans, span_indices, label_lst, target_indices, opinion_indices, distance_embeddings):
    B, N, H = spans.shape
    T = target_indices.shape[1]
    O = opinion_indices.shape[1]
    P = T * O
    D = distance_embeddings.shape[1]
    W = 2 * H + D
    NB = 16

    spans2d = spans.reshape(B * N, H)
    t_flat = target_indices.reshape(-1)
    o_flat = opinion_indices.reshape(-1)
    span_start = span_indices[:, 0]
    span_end = span_indices[:, 1]
    demb16 = jnp.zeros((NB, D), jnp.float32).at[:14].set(distance_embeddings)

    mesh = plsc.VectorSubcoreMesh(core_axis_name="c", subcore_axis_name="s")
    TH = T // 2

    @functools.partial(
        pl.kernel,
        mesh=mesh,
        compiler_params=pltpu.CompilerParams(needs_layout_passes=False),
        out_type=(
            jax.ShapeDtypeStruct((B * (T + O), H), jnp.float32),
            jax.ShapeDtypeStruct((B, NB, P), jnp.float32),
        ),
        scratch_types=[
            pltpu.VMEM((TH,), jnp.int32),
            pltpu.VMEM((O,), jnp.int32),
            pltpu.VMEM((TH,), jnp.int32),
            pltpu.VMEM((TH,), jnp.int32),
            pltpu.VMEM((N,), jnp.int32),
            pltpu.VMEM((N,), jnp.int32),
            pltpu.VMEM((TH, H), jnp.float32),
            pltpu.VMEM((TH, H), jnp.float32),
            pltpu.VMEM((NB, TH * O), jnp.float32),
            pltpu.SemaphoreType.DMA,
        ],
    )
    def sc_kern(spans_hbm, sstart_hbm, send_hbm, tflat_hbm, oflat_hbm,
                compact_hbm, emoh_hbm,
                tind_v, oind_v, tidx_v, oidx_v, sstart_v, send_v,
                trows_v, orows_v, emoh_v,
                sem_in):
        b = lax.axis_index("s")
        h = lax.axis_index("c")
        t0 = h * TH
        base = b * N

        h_in = [
            pltpu.async_copy(tflat_hbm.at[pl.ds(b * T + t0, TH)], tind_v, sem_in),
            pltpu.async_copy(oflat_hbm.at[pl.ds(b * O, O)], oind_v, sem_in),
            pltpu.async_copy(sstart_hbm, sstart_v, sem_in),
            pltpu.async_copy(send_hbm, send_v, sem_in),
        ]
        for hd in h_in:
            hd.wait()

        iota = lax.iota(jnp.int32, 16)
        tidx_v[pl.ds(0, 16)] = tind_v[pl.ds(0, 16)] + base
        oidx_v[pl.ds(0, 16)] = oind_v[pl.ds(t0, 16)] + base
        g1 = pltpu.async_copy(spans_hbm.at[tidx_v], trows_v, sem_in)
        g2 = pltpu.async_copy(spans_hbm.at[oidx_v], orows_v, sem_in)

        o_raw = [oind_v[pl.ds(half * 16, 16)] for half in range(O // 16)]
        c_o = [plsc.load_gather(sstart_v, [o_raw[half]]) for half in range(O // 16)]
        d_o = [plsc.load_gather(send_v, [o_raw[half]]) for half in range(O // 16)]
        tvec = tind_v[pl.ds(0, 16)]

        for t in range(TH):
            tt_s = jnp.sum(tvec * (iota == t).astype(jnp.int32))
            tt_raw = jnp.zeros((16,), jnp.int32) + tt_s
            a_t = plsc.load_gather(sstart_v, [tt_raw])
            b_t = plsc.load_gather(send_v, [tt_raw])
            for half in range(O // 16):
                width = jnp.minimum(jnp.abs(b_t - c_o[half]), jnp.abs(a_t - d_o[half]))
                em = jnp.full((16,), -1, jnp.int32)
                for edge in _BINS:
                    em = em + (width >= edge).astype(jnp.int32)
                lp = t * O + half * 16
                for j in range(NB - 2):
                    emoh_v[j, pl.ds(lp, 16)] = (em == j).astype(jnp.float32)
                for j in (NB - 2, NB - 1):
                    emoh_v[j, pl.ds(lp, 16)] = jnp.zeros((16,), jnp.float32)
        pltpu.sync_copy(emoh_v, emoh_hbm.at[b, :, pl.ds(t0 * O, TH * O)])
        g1.wait()
        g2.wait()
        crow = b * (T + O)
        pltpu.sync_copy(trows_v, compact_hbm.at[pl.ds(crow + t0, TH)])
        pltpu.sync_copy(orows_v, compact_hbm.at[pl.ds(crow + T + t0, TH)])

    compact2d, emohT = sc_kern(spans2d, span_start, span_end, t_flat, o_flat)
    t3 = target_indices.reshape(B, 1, T)
    o3 = opinion_indices.reshape(B, 1, O)

    def tc_body(compact_ref, emoh_ref, demb_ref, t_ref, o_ref, out_ref, rel_ref):
        orows = compact_ref[T:T + O, :]
        for t in range(T):
            r0 = t * O
            tgt = compact_ref[pl.ds(t, 1), :]
            out_ref[0, r0:r0 + O, 0:H] = jnp.broadcast_to(tgt, (O, H))
            out_ref[0, r0:r0 + O, H:2 * H] = orows
        out_ref[0, :, 2 * H:W] = lax.dot_general(
            emoh_ref[0], demb_ref[...],
            dimension_numbers=(((0,), (0,)), ((), ())),
            preferred_element_type=jnp.float32)
        pid = lax.broadcasted_iota(jnp.int32, (P, T), 0)
        tsel = (pid // O == lax.broadcasted_iota(jnp.int32, (P, T), 1))
        osel = (lax.broadcasted_iota(jnp.int32, (P, O), 0) % O ==
                lax.broadcasted_iota(jnp.int32, (P, O), 1))
        tv2 = t_ref[0, :, :].astype(jnp.float32)
        ov2 = o_ref[0, :, :].astype(jnp.float32)
        trep = lax.dot_general(tsel.astype(jnp.float32), tv2,
                               dimension_numbers=(((1,), (1,)), ((), ())),
                               preferred_element_type=jnp.float32)
        otile = lax.dot_general(osel.astype(jnp.float32), ov2,
                                dimension_numbers=(((1,), (1,)), ((), ())),
                                preferred_element_type=jnp.float32)
        rel_ref[0, :, :] = jnp.concatenate(
            [trep, otile], axis=1).astype(jnp.int32)

    out3d, rel3d = pl.pallas_call(
        tc_body,
        grid=(B,),
        in_specs=[
            pl.BlockSpec((T + O, H), lambda s: (s, 0)),
            pl.BlockSpec((1, NB, P), lambda s: (s, 0, 0)),
            pl.BlockSpec((NB, D), lambda s: (0, 0)),
            pl.BlockSpec((1, 1, T), lambda s: (s, 0, 0)),
            pl.BlockSpec((1, 1, O), lambda s: (s, 0, 0)),
        ],
        out_specs=(
            pl.BlockSpec((1, P, W), lambda s: (s, 0, 0)),
            pl.BlockSpec((1, P, 2), lambda s: (s, 0, 0)),
        ),
        out_shape=(
            jax.ShapeDtypeStruct((B, P, W), jnp.float32),
            jax.ShapeDtypeStruct((B, P, 2), jnp.int32),
        ),
        compiler_params=pltpu.CompilerParams(
            dimension_semantics=("arbitrary",),
        ),
    )(compact2d, emohT, demb16, t3, o3)

    return out3d, rel3d
